# baseline (speedup 1.0000x reference)
import functools

import jax
import jax.numpy as jnp
from jax.experimental import pallas as pl
from jax.experimental.pallas import tpu as pltpu

_HID = 32
_INIT_LEN = 4
_F32 = jnp.float32
_BF = jnp.bfloat16


def _step_kernel(a_ref, xc_ref, xn_ref, w0_ref, w1_ref, w2_ref, wm_ref,
                 wih_ref, whh_ref, wo_ref, b0_ref, b1_ref, b2_ref, bm_ref,
                 bih_ref, bhh_ref, bo_ref,
                 out_ref, hid_ref, wk0_s, wk12_s, wkm_s, wgi_s, wgh_s, wok_s,
                 bias_s, xs_s, xn_s):
    i = pl.program_id(0)
    A = a_ref[0]
    C = A.shape[0]
    H = _HID
    B = out_ref.shape[0]
    G = B * H

    @pl.when(i == 0)
    def _prep_weights():
        wk0_s[...] = jnp.zeros(wk0_s.shape, _BF)
        wk12_s[...] = jnp.zeros(wk12_s.shape, _BF)
        wkm_s[...] = jnp.zeros(wkm_s.shape, _BF)
        wgi_s[...] = jnp.zeros(wgi_s.shape, _BF)
        wgh_s[...] = jnp.zeros(wgh_s.shape, _BF)
        wok_s[...] = jnp.zeros(wok_s.shape, _BF)
        w0 = w0_ref[...].astype(_BF)
        w1 = w1_ref[...].astype(_BF)
        w2 = w2_ref[...].astype(_BF)
        wm = wm_ref[...].astype(_BF)
        wih = wih_ref[...].astype(_BF)
        whh = whh_ref[...].astype(_BF)
        wo = wo_ref[...].astype(_BF)
        F = w0.shape[0]
        for b in range(B):
            for j in range(F):
                wk0_s[j * B + b:j * B + b + 1, b * H:(b + 1) * H] = (
                    w0[j:j + 1, :])
                for g in range(3):
                    wgi_s[j * B + b:j * B + b + 1,
                          g * G + b * H:g * G + (b + 1) * H] = (
                        wih[g * H:(g + 1) * H, j:j + 1].T)
            wk12_s[b * H:(b + 1) * H, b * H:(b + 1) * H] = w1
            wk12_s[b * H:(b + 1) * H, G + b * H:G + (b + 1) * H] = w2
            wkm_s[b * H:(b + 1) * H, b * H:(b + 1) * H] = wm[:H]
            wkm_s[G + b * H:G + (b + 1) * H, b * H:(b + 1) * H] = wm[H:]
            wok_s[b * H:(b + 1) * H, b:b + 1] = wo
            for g in range(3):
                wgh_s[b * H:(b + 1) * H, g * G + b * H:g * G + (b + 1) * H] = (
                    whh[g * H:(g + 1) * H, :].T)
        bias_s[0:1, :G] = jnp.tile(b0_ref[...], (1, B))
        bias_s[1:2, :G] = jnp.tile(b1_ref[...], (1, B))
        bias_s[2:3, :G] = jnp.tile(b2_ref[...], (1, B))
        bias_s[3:4, :G] = jnp.tile(bm_ref[...], (1, B))
        for g in range(3):
            bias_s[4:5, g * G:(g + 1) * G] = jnp.tile(
                bih_ref[:, g * H:(g + 1) * H], (1, B))
            bias_s[5:6, g * G:(g + 1) * G] = jnp.tile(
                bhh_ref[:, g * H:(g + 1) * H], (1, B))
        bias_s[6:7, :B] = jnp.tile(bo_ref[...], (1, B))

    Ab = A.astype(_BF)

    d_r = jnp.sum(A, axis=1, keepdims=True) + 1.0
    d_c = jnp.sum(A, axis=0, keepdims=True).T + 1.0
    dinv_r = jax.lax.rsqrt(d_r)
    dinv_c = jax.lax.rsqrt(d_c)

    def norm_matmul_f(Z):
        Zs = dinv_r * Z
        AZ = jnp.dot(Ab, Zs.astype(_BF), preferred_element_type=_F32)
        return dinv_r * (AZ + Zs)

    def norm_matmul_b(Z):
        Zs = dinv_c * Z
        At_Zs = jax.lax.dot_general(Ab, Zs.astype(_BF),
                                    (((0,), (0,)), ((), ())),
                                    preferred_element_type=_F32)
        return dinv_c * (At_Zs + Zs)

    b0k = bias_s[0:1, :G]
    b1k = bias_s[1:2, :G]
    b2k = bias_s[2:3, :G]
    bmk = bias_s[3:4, :G]

    Fin = w0_ref.shape[0]
    xs_s[...] = xc_ref[0].T
    x_t_f32 = jnp.concatenate([xs_s[j::Fin, :] for j in range(Fin)],
                              axis=1)

    @pl.when(i == 0)
    def _init():
        y0 = jnp.dot(x_t_f32.astype(_BF), wk0_s[...],
                     preferred_element_type=_F32)
        hid_ref[...] = jax.nn.relu(norm_matmul_f(y0) + b0k)

    h = hid_ref[...]

    y = jnp.dot(h.astype(_BF), wk12_s[...],
                preferred_element_type=_F32)
    fh = jax.nn.relu(norm_matmul_f(y[:, :G]) + b1k)
    bh = jax.nn.relu(norm_matmul_b(y[:, G:]) + b2k)

    hs = (jnp.dot(jnp.concatenate([fh, bh], axis=1).astype(_BF), wkm_s[...],
                  preferred_element_type=_F32)
          + bmk)

    x_t = x_t_f32.astype(_BF)
    gi = jnp.dot(x_t, wgi_s[...], preferred_element_type=_F32) + bias_s[4:5, :]
    gh = jnp.dot(hs.astype(_BF), wgh_s[...],
                 preferred_element_type=_F32) + bias_s[5:6, :]

    r = jax.nn.sigmoid(gi[:, :G] + gh[:, :G])
    z = jax.nn.sigmoid(gi[:, G:2 * G] + gh[:, G:2 * G])
    n = jnp.tanh(gi[:, 2 * G:] + r * gh[:, 2 * G:])
    h_new = (1.0 - z) * n + z * hs
    hid_ref[...] = h_new

    @pl.when(i >= _INIT_LEN)
    def _emit():
        half = C // 2
        pred = (jnp.dot(h_new[half:].astype(_BF), wok_s[...],
                        preferred_element_type=_F32)
                + bias_s[6:7, :B])
        xn_s[...] = xn_ref[0, :, :C].T
        obs = xn_s[::Fin, :][:half, :]
        out_ref[:, i - _INIT_LEN, :half] = obs.T
        out_ref[:, i - _INIT_LEN, half:] = pred.T


@functools.partial(jax.jit, static_argnums=())
def kernel(input_data, adj_list, W0, b0, W1, b1, W2, b2, Wm, bm,
           W_ih, W_hh, b_ih, b_hh, Wo, bo):
    B, T, C, F = input_data.shape
    H = _HID
    G = B * H

    xflat = jnp.transpose(input_data, (1, 0, 2, 3)).reshape(T, B, C * F)

    row = lambda v: v.reshape(1, -1)

    num_steps = T - 1
    num_out = num_steps - _INIT_LEN

    full = lambda shape: pl.BlockSpec(shape, lambda i: (0,) * len(shape))

    out = pl.pallas_call(
        _step_kernel,
        grid=(num_steps,),
        in_specs=[
            pl.BlockSpec((1, C, C), lambda i: (i, 0, 0)),
            pl.BlockSpec((1, B, C * F), lambda i: (i, 0, 0)),
            pl.BlockSpec((1, B, C * F), lambda i: (i + 1, 0, 0)),
            full(W0.shape), full(W1.shape), full(W2.shape), full(Wm.shape),
            full(W_ih.shape), full(W_hh.shape), full(Wo.shape),
            full((1, H)), full((1, H)), full((1, H)), full((1, H)),
            full((1, 3 * H)), full((1, 3 * H)), full((1, 1)),
        ],
        out_specs=full((B, num_out, C)),
        out_shape=jax.ShapeDtypeStruct((B, num_out, C), _F32),
        scratch_shapes=[
            pltpu.VMEM((C, G), _F32),
            pltpu.VMEM((B * F, G), _BF),
            pltpu.VMEM((G, 2 * G), _BF),
            pltpu.VMEM((2 * G, G), _BF),
            pltpu.VMEM((B * F, 3 * G), _BF),
            pltpu.VMEM((G, 3 * G), _BF),
            pltpu.VMEM((G, B), _BF),
            pltpu.VMEM((8, 3 * G), _F32),
            pltpu.VMEM((C * F, B), _F32),
            pltpu.VMEM((C, B), _F32),
        ],
        compiler_params=pltpu.CompilerParams(
            dimension_semantics=("arbitrary",)),
    )(adj_list, xflat, xflat, W0, W1, W2, Wm, W_ih, W_hh, Wo,
      row(b0), row(b1), row(b2), row(bm), row(b_ih), row(b_hh), row(bo))

    return out[..., None]


# speedup vs baseline: 1.1146x; 1.1146x over previous
import functools

import jax
import jax.numpy as jnp
from jax.experimental import pallas as pl
from jax.experimental.pallas import tpu as pltpu

_HID = 32
_INIT_LEN = 4
_F32 = jnp.float32
_BF = jnp.bfloat16


def _step_kernel(a_ref, x_ref, w0_ref, w1_ref, w2_ref, wm_ref, wih_ref,
                 whh_ref, wo_ref, b0_ref, b1_ref, b2_ref, bm_ref,
                 bih_ref, bhh_ref, bo_ref,
                 out_ref, hid_ref, wk0_s, wk12_s, wkm_s, wgi_s, wgh_s, wok_s,
                 bias_s, sel_s):
    i = pl.program_id(0)
    A = a_ref[0]
    C = A.shape[0]
    H = _HID
    B = out_ref.shape[0]
    G = B * H

    @pl.when(i == 0)
    def _prep_weights():
        wk0_s[...] = jnp.zeros(wk0_s.shape, _BF)
        wk12_s[...] = jnp.zeros(wk12_s.shape, _BF)
        wkm_s[...] = jnp.zeros(wkm_s.shape, _BF)
        wgi_s[...] = jnp.zeros(wgi_s.shape, _BF)
        wgh_s[...] = jnp.zeros(wgh_s.shape, _BF)
        wok_s[...] = jnp.zeros(wok_s.shape, _BF)
        w0 = w0_ref[...].astype(_BF)
        w1 = w1_ref[...].astype(_BF)
        w2 = w2_ref[...].astype(_BF)
        wm = wm_ref[...].astype(_BF)
        wih = wih_ref[...].astype(_BF)
        whh = whh_ref[...].astype(_BF)
        wo = wo_ref[...].astype(_BF)
        F = w0.shape[0]
        for b in range(B):
            wk0_s[b * F:(b + 1) * F, b * H:(b + 1) * H] = w0
            wk12_s[b * H:(b + 1) * H, b * H:(b + 1) * H] = w1
            wk12_s[b * H:(b + 1) * H, G + b * H:G + (b + 1) * H] = w2
            wkm_s[b * H:(b + 1) * H, b * H:(b + 1) * H] = wm[:H]
            wkm_s[G + b * H:G + (b + 1) * H, b * H:(b + 1) * H] = wm[H:]
            wok_s[b * H:(b + 1) * H, b:b + 1] = wo
            for g in range(3):
                wgi_s[b * F:(b + 1) * F, g * G + b * H:g * G + (b + 1) * H] = (
                    wih[g * H:(g + 1) * H, :].T)
                wgh_s[b * H:(b + 1) * H, g * G + b * H:g * G + (b + 1) * H] = (
                    whh[g * H:(g + 1) * H, :].T)
        bias_s[0:1, :G] = jnp.tile(b0_ref[...], (1, B))
        bias_s[1:2, :G] = jnp.tile(b1_ref[...], (1, B))
        bias_s[2:3, :G] = jnp.tile(b2_ref[...], (1, B))
        bias_s[3:4, :G] = jnp.tile(bm_ref[...], (1, B))
        for g in range(3):
            bias_s[4:5, g * G:(g + 1) * G] = jnp.tile(
                bih_ref[:, g * H:(g + 1) * H], (1, B))
            bias_s[5:6, g * G:(g + 1) * G] = jnp.tile(
                bhh_ref[:, g * H:(g + 1) * H], (1, B))
        bias_s[6:7, :B] = jnp.tile(bo_ref[...], (1, B))
        F2 = w0_ref.shape[0]
        rr = jax.lax.broadcasted_iota(jnp.int32, sel_s.shape, 0)
        cc = jax.lax.broadcasted_iota(jnp.int32, sel_s.shape, 1)
        sel_s[...] = jnp.where(rr == cc * F2, 1.0, 0.0).astype(_F32)

    Ab = A.astype(_BF)

    d_r = jnp.sum(A, axis=1, keepdims=True) + 1.0
    d_c = jnp.sum(A, axis=0, keepdims=True).T + 1.0
    dinv_r = jax.lax.rsqrt(d_r)
    dinv_c = jax.lax.rsqrt(d_c)

    def norm_matmul_f(Z):
        Zs = dinv_r * Z
        AZ = jnp.dot(Ab, Zs.astype(_BF), preferred_element_type=_F32)
        return dinv_r * (AZ + Zs)

    def norm_matmul_b(Z):
        Zs = dinv_c * Z
        At_Zs = jax.lax.dot_general(Ab, Zs.astype(_BF),
                                    (((0,), (0,)), ((), ())),
                                    preferred_element_type=_F32)
        return dinv_c * (At_Zs + Zs)

    b0k = bias_s[0:1, :G]
    b1k = bias_s[1:2, :G]
    b2k = bias_s[2:3, :G]
    bmk = bias_s[3:4, :G]

    @pl.when(i == 0)
    def _init():
        x0 = x_ref[0].astype(_BF)
        y0 = jnp.dot(x0, wk0_s[...], preferred_element_type=_F32)
        hid_ref[...] = jax.nn.relu(norm_matmul_f(y0) + b0k)

    h = hid_ref[...]

    y = jnp.dot(h.astype(_BF), wk12_s[...],
                preferred_element_type=_F32)
    fh = jax.nn.relu(norm_matmul_f(y[:, :G]) + b1k)
    bh = jax.nn.relu(norm_matmul_b(y[:, G:]) + b2k)

    hs = (jnp.dot(jnp.concatenate([fh, bh], axis=1).astype(_BF), wkm_s[...],
                  preferred_element_type=_F32)
          + bmk)

    x_t = x_ref[i].astype(_BF)
    gi = jnp.dot(x_t, wgi_s[...], preferred_element_type=_F32) + bias_s[4:5, :]
    gh = jnp.dot(hs.astype(_BF), wgh_s[...],
                 preferred_element_type=_F32) + bias_s[5:6, :]

    r = jax.nn.sigmoid(gi[:, :G] + gh[:, :G])
    z = jax.nn.sigmoid(gi[:, G:2 * G] + gh[:, G:2 * G])
    n = jnp.tanh(gi[:, 2 * G:] + r * gh[:, 2 * G:])
    h_new = (1.0 - z) * n + z * hs
    hid_ref[...] = h_new

    @pl.when(i >= _INIT_LEN)
    def _emit():
        half = C // 2
        pred = (jnp.dot(h_new[half:].astype(_BF), wok_s[...],
                        preferred_element_type=_F32)
                + bias_s[6:7, :B])
        obs = jnp.dot(x_ref[i + 1, :half, :], sel_s[...],
                      precision=jax.lax.Precision.HIGHEST,
                      preferred_element_type=_F32)
        out_ref[:, i - _INIT_LEN, :half] = obs.T
        out_ref[:, i - _INIT_LEN, half:] = pred.T


@functools.partial(jax.jit, static_argnums=())
def kernel(input_data, adj_list, W0, b0, W1, b1, W2, b2, Wm, bm,
           W_ih, W_hh, b_ih, b_hh, Wo, bo):
    B, T, C, F = input_data.shape
    H = _HID
    G = B * H

    xT = jnp.transpose(input_data, (1, 2, 0, 3)).reshape(T, C, B * F)

    row = lambda v: v.reshape(1, -1)

    num_steps = T - 1
    num_out = num_steps - _INIT_LEN

    full = lambda shape: pl.BlockSpec(shape, lambda i: (0,) * len(shape))

    out = pl.pallas_call(
        _step_kernel,
        grid=(num_steps,),
        in_specs=[
            pl.BlockSpec((1, C, C), lambda i: (i, 0, 0)),
            full((T, C, B * F)),
            full(W0.shape), full(W1.shape), full(W2.shape), full(Wm.shape),
            full(W_ih.shape), full(W_hh.shape), full(Wo.shape),
            full((1, H)), full((1, H)), full((1, H)), full((1, H)),
            full((1, 3 * H)), full((1, 3 * H)), full((1, 1)),
        ],
        out_specs=full((B, num_out, C)),
        out_shape=jax.ShapeDtypeStruct((B, num_out, C), _F32),
        scratch_shapes=[
            pltpu.VMEM((C, G), _F32),
            pltpu.VMEM((B * F, G), _BF),
            pltpu.VMEM((G, 2 * G), _BF),
            pltpu.VMEM((2 * G, G), _BF),
            pltpu.VMEM((B * F, 3 * G), _BF),
            pltpu.VMEM((G, 3 * G), _BF),
            pltpu.VMEM((G, B), _BF),
            pltpu.VMEM((8, 3 * G), _F32),
            pltpu.VMEM((B * F, B), _F32),
        ],
        compiler_params=pltpu.CompilerParams(
            dimension_semantics=("arbitrary",)),
    )(adj_list, xT, W0, W1, W2, Wm, W_ih, W_hh, Wo,
      row(b0), row(b1), row(b2), row(bm), row(b_ih), row(b_hh), row(bo))

    return out[..., None]
